# baseline (speedup 1.0000x reference)
import functools

import jax
import jax.numpy as jnp
from jax import lax
from jax.experimental import pallas as pl
from jax.experimental.pallas import tpu as pltpu
from jax.experimental.pallas import tpu_sc as plsc

B, N = 1, 6
C, H, W = 64, 32, 88
GH, GW = 128, 128
K = GH * GW
NGP = 9
IMG_H, IMG_W = 256.0, 704.0
EPS = 1e-05
BEV_Z = -1.0

NC, NS = 2, 16
CPC = N // NC
TAB_ROWS = CPC * H * W
STAGE_ROWS = TAB_ROWS // NS
PPT = K // NS
P_CHUNK = 512
SUB = 128
NSUB = P_CHUNK // SUB
N_UNITS = CPC * NGP
CPU_ = PPT // P_CHUNK
N_CHUNKS = N_UNITS * CPU_
IDX_ROWS = N_CHUNKS * NSUB
NBUF = 2
STEADY = (N_CHUNKS - NBUF) // NBUF


def _tc_index_body(i_ref, e_ref, p_ref, img_ref, inds_ref, mask_ref, tab_ref):
    n = pl.program_id(0)
    tab_ref[0] = img_ref[0].T
    p2i = jnp.matmul(i_ref[0], e_ref[0])
    sp = lax.dot_general(p2i, p_ref[...], (((1,), (1,)), ((), ())))
    sx, sy, sz = sp[0:1], sp[1:2], sp[2:3]

    zed = jnp.maximum(sz, EPS)
    xn = sx / zed / IMG_W
    yn = sy / zed / IMG_H
    mask_ref[0] = ((sz > EPS)
                   & (xn > 0.0) & (xn < 1.0)
                   & (yn > 0.0) & (yn < 1.0))

    u = jnp.round(xn * float(W))
    v = jnp.round(yn * float(H))
    cam_off = (lax.rem(n, CPC) * (H * W)).astype(jnp.float32)
    for g in range(NGP):
        dx = float(g % 3 - 1)
        dy = float(g // 3 - 1)
        xi = jnp.clip(u + dx, 0.0, float(W - 1))
        yi = jnp.clip(v + dy, 0.0, float(H - 1))
        ind = xi + yi * float(W) + cam_off
        inds_ref[0, :, 0, g] = ind.astype(jnp.int32).reshape(NS, CPU_ * NSUB, SUB)


def _tc_indices(intrin, e_mats, pts4, images):
    return pl.pallas_call(
        _tc_index_body,
        grid=(N,),
        in_specs=[
            pl.BlockSpec((1, 4, 4), lambda n: (n, 0, 0)),
            pl.BlockSpec((1, 4, 4), lambda n: (n, 0, 0)),
            pl.BlockSpec((K, 4), lambda n: (0, 0)),
            pl.BlockSpec((1, C, H * W), lambda n: (n, 0, 0)),
        ],
        out_specs=[
            pl.BlockSpec((1, NS, 1, NGP, CPU_ * NSUB, SUB),
                         lambda n: (n // CPC, 0, n % CPC, 0, 0, 0)),
            pl.BlockSpec((1, 1, K), lambda n: (n, 0, 0)),
            pl.BlockSpec((1, H * W, C), lambda n: (n, 0, 0)),
        ],
        out_shape=[
            jax.ShapeDtypeStruct((NC, NS, CPC, NGP, CPU_ * NSUB, SUB),
                                 jnp.int32),
            jax.ShapeDtypeStruct((N, 1, K), jnp.bool_),
            jax.ShapeDtypeStruct((N, H * W, C), jnp.float32),
        ],
    )(intrin, e_mats, pts4, images)


def _sc_gather_body(tab_hbm, inds_hbm, out_hbm,
                    tab_sp, idx_v, b0, b1, g0, g1, w0, w1):
    cid = lax.axis_index("c")
    sid = lax.axis_index("s")
    bufs = (b0, b1)
    gsems = (g0, g1)
    wsems = (w0, w1)

    pltpu.sync_copy(
        tab_hbm.at[pl.ds(cid * TAB_ROWS + sid * STAGE_ROWS, STAGE_ROWS)],
        tab_sp.at[pl.ds(sid * STAGE_ROWS, STAGE_ROWS)])
    pltpu.sync_copy(inds_hbm.at[cid, sid], idx_v)
    plsc.subcore_barrier()

    def fire_gather(t, nb):
        for j in range(NSUB):
            pltpu.async_copy(tab_sp.at[idx_v.at[t * NSUB + j]],
                             bufs[nb].at[pl.ds(j * SUB, SUB)], gsems[nb])

    def drain(sem, buf):
        pltpu.make_async_copy(tab_hbm.at[pl.ds(0, P_CHUNK)], buf, sem).wait()

    def retire_chunk(t, nb):
        drain(gsems[nb], bufs[nb])
        cam = t // (NGP * CPU_)
        rem = lax.rem(t, NGP * CPU_)
        g = rem // CPU_
        sub = lax.rem(rem, CPU_)
        p0 = sid * PPT + sub * P_CHUNK
        pltpu.async_copy(
            bufs[nb],
            out_hbm.at[0, cid * CPC + cam, pl.ds(p0, P_CHUNK), g],
            wsems[nb])
        drain(wsems[nb], bufs[nb])

    for nb in range(NBUF):
        fire_gather(nb, nb)

    @pl.loop(0, STEADY)
    def _steady(o):
        for nb in range(NBUF):
            t = o * NBUF + nb
            retire_chunk(t, nb)
            fire_gather(t + NBUF, nb)

    for nb in range(NBUF):
        retire_chunk(N_CHUNKS - NBUF + nb, nb)


def _sc_gather(imtab, inds):
    call = pl.kernel(
        _sc_gather_body,
        out_type=jax.ShapeDtypeStruct((B, N, K, NGP, C), jnp.float32),
        mesh=plsc.VectorSubcoreMesh(core_axis_name="c", subcore_axis_name="s",
                                    num_cores=NC, num_subcores=NS),
        scratch_types=[
            pltpu.VMEM_SHARED((TAB_ROWS, C), jnp.float32),
            pltpu.VMEM((IDX_ROWS, SUB), jnp.int32),
            pltpu.VMEM((P_CHUNK, C), jnp.float32),
            pltpu.VMEM((P_CHUNK, C), jnp.float32),
            pltpu.SemaphoreType.DMA,
            pltpu.SemaphoreType.DMA,
            pltpu.SemaphoreType.DMA,
            pltpu.SemaphoreType.DMA,
        ],
        compiler_params=pltpu.CompilerParams(use_tc_tiling_on_sc=False),
    )
    return call(imtab, inds)


def kernel(bev_grids, images, I, E):
    intrin = jnp.pad(I, ((0, 0), (0, 0), (0, 1), (0, 1)))
    intrin = intrin.at[..., 3, 3].set(1.0)
    xc = bev_grids[0].reshape(K, 1)
    yc = bev_grids[1].reshape(K, 1)
    pts4 = jnp.concatenate(
        [xc, yc, jnp.full_like(xc, BEV_Z), jnp.ones_like(xc)], axis=1)
    inds, mask, imtab = _tc_indices(intrin[0], E.reshape(N, 4, 4), pts4,
                                    images.reshape(N, C, H * W))
    idx_w = inds.reshape(NC, NS, IDX_ROWS, SUB)
    feats = _sc_gather(imtab.reshape(N * H * W, C), idx_w)
    return feats, mask.reshape(B, N, K, 1)


# speedup vs baseline: 1.0280x; 1.0280x over previous
import functools

import jax
import jax.numpy as jnp
from jax import lax
from jax.experimental import pallas as pl
from jax.experimental.pallas import tpu as pltpu
from jax.experimental.pallas import tpu_sc as plsc

B, N = 1, 6
C, H, W = 64, 32, 88
GH, GW = 128, 128
K = GH * GW
NGP = 9
IMG_H, IMG_W = 256.0, 704.0
EPS = 1e-05
BEV_Z = -1.0

NC, NS = 2, 16
CPC = N // NC
TAB_ROWS = CPC * H * W
STAGE_ROWS = TAB_ROWS // NS
PPT = K // NS
P_CHUNK = 256
SUB = 128
NSUB = P_CHUNK // SUB
N_UNITS = CPC * NGP
CPU_ = PPT // P_CHUNK
N_CHUNKS = N_UNITS * CPU_
IDX_ROWS = N_CHUNKS * NSUB
NBUF = 3
STEADY = (N_CHUNKS - NBUF) // NBUF


def _tc_index_body(i_ref, e_ref, p_ref, img_ref, inds_ref, mask_ref, tab_ref):
    n = pl.program_id(0)
    tab_ref[0] = img_ref[0].T
    p2i = jnp.matmul(i_ref[0], e_ref[0])
    sp = lax.dot_general(p2i, p_ref[...], (((1,), (1,)), ((), ())))
    sx, sy, sz = sp[0:1], sp[1:2], sp[2:3]

    zed = jnp.maximum(sz, EPS)
    xn = sx / zed / IMG_W
    yn = sy / zed / IMG_H
    mask_ref[0] = ((sz > EPS)
                   & (xn > 0.0) & (xn < 1.0)
                   & (yn > 0.0) & (yn < 1.0))

    u = jnp.round(xn * float(W))
    v = jnp.round(yn * float(H))
    cam_off = (lax.rem(n, CPC) * (H * W)).astype(jnp.float32)
    for g in range(NGP):
        dx = float(g % 3 - 1)
        dy = float(g // 3 - 1)
        xi = jnp.clip(u + dx, 0.0, float(W - 1))
        yi = jnp.clip(v + dy, 0.0, float(H - 1))
        ind = xi + yi * float(W) + cam_off
        inds_ref[0, :, 0, g] = ind.astype(jnp.int32).reshape(NS, CPU_ * NSUB, SUB)


def _tc_indices(intrin, e_mats, pts4, images):
    return pl.pallas_call(
        _tc_index_body,
        grid=(N,),
        in_specs=[
            pl.BlockSpec((1, 4, 4), lambda n: (n, 0, 0)),
            pl.BlockSpec((1, 4, 4), lambda n: (n, 0, 0)),
            pl.BlockSpec((K, 4), lambda n: (0, 0)),
            pl.BlockSpec((1, C, H * W), lambda n: (n, 0, 0)),
        ],
        out_specs=[
            pl.BlockSpec((1, NS, 1, NGP, CPU_ * NSUB, SUB),
                         lambda n: (n // CPC, 0, n % CPC, 0, 0, 0)),
            pl.BlockSpec((1, 1, K), lambda n: (n, 0, 0)),
            pl.BlockSpec((1, H * W, C), lambda n: (n, 0, 0)),
        ],
        out_shape=[
            jax.ShapeDtypeStruct((NC, NS, CPC, NGP, CPU_ * NSUB, SUB),
                                 jnp.int32),
            jax.ShapeDtypeStruct((N, 1, K), jnp.bool_),
            jax.ShapeDtypeStruct((N, H * W, C), jnp.float32),
        ],
    )(intrin, e_mats, pts4, images)


def _sc_gather_body(tab_hbm, inds_hbm, out_hbm,
                    tab_sp, idx_v, b0, b1, b2, g0, g1, g2, w0, w1, w2):
    cid = lax.axis_index("c")
    sid = lax.axis_index("s")
    bufs = (b0, b1, b2)
    gsems = (g0, g1, g2)
    wsems = (w0, w1, w2)

    pltpu.sync_copy(
        tab_hbm.at[pl.ds(cid * TAB_ROWS + sid * STAGE_ROWS, STAGE_ROWS)],
        tab_sp.at[pl.ds(sid * STAGE_ROWS, STAGE_ROWS)])
    pltpu.sync_copy(inds_hbm.at[cid, sid], idx_v)
    plsc.subcore_barrier()

    def fire_gather(t, nb):
        for j in range(NSUB):
            pltpu.async_copy(tab_sp.at[idx_v.at[t * NSUB + j]],
                             bufs[nb].at[pl.ds(j * SUB, SUB)], gsems[nb])

    def drain(sem, buf):
        pltpu.make_async_copy(tab_hbm.at[pl.ds(0, P_CHUNK)], buf, sem).wait()

    def retire_chunk(t, nb):
        drain(gsems[nb], bufs[nb])
        cam = t // (NGP * CPU_)
        rem = lax.rem(t, NGP * CPU_)
        g = rem // CPU_
        sub = lax.rem(rem, CPU_)
        p0 = sid * PPT + sub * P_CHUNK
        pltpu.async_copy(
            bufs[nb],
            out_hbm.at[0, cid * CPC + cam, pl.ds(p0, P_CHUNK), g],
            wsems[nb])
        drain(wsems[nb], bufs[nb])

    for nb in range(NBUF):
        fire_gather(nb, nb)

    @pl.loop(0, STEADY)
    def _steady(o):
        for nb in range(NBUF):
            t = o * NBUF + nb
            retire_chunk(t, nb)
            fire_gather(t + NBUF, nb)

    for nb in range(NBUF):
        retire_chunk(N_CHUNKS - NBUF + nb, nb)


def _sc_gather(imtab, inds):
    call = pl.kernel(
        _sc_gather_body,
        out_type=jax.ShapeDtypeStruct((B, N, K, NGP, C), jnp.float32),
        mesh=plsc.VectorSubcoreMesh(core_axis_name="c", subcore_axis_name="s",
                                    num_cores=NC, num_subcores=NS),
        scratch_types=[
            pltpu.VMEM_SHARED((TAB_ROWS, C), jnp.float32),
            pltpu.VMEM((IDX_ROWS, SUB), jnp.int32),
            pltpu.VMEM((P_CHUNK, C), jnp.float32),
            pltpu.VMEM((P_CHUNK, C), jnp.float32),
            pltpu.VMEM((P_CHUNK, C), jnp.float32),
            pltpu.SemaphoreType.DMA,
            pltpu.SemaphoreType.DMA,
            pltpu.SemaphoreType.DMA,
            pltpu.SemaphoreType.DMA,
            pltpu.SemaphoreType.DMA,
            pltpu.SemaphoreType.DMA,
        ],
        compiler_params=pltpu.CompilerParams(use_tc_tiling_on_sc=False),
    )
    return call(imtab, inds)


def kernel(bev_grids, images, I, E):
    intrin = jnp.pad(I, ((0, 0), (0, 0), (0, 1), (0, 1)))
    intrin = intrin.at[..., 3, 3].set(1.0)
    xc = bev_grids[0].reshape(K, 1)
    yc = bev_grids[1].reshape(K, 1)
    pts4 = jnp.concatenate(
        [xc, yc, jnp.full_like(xc, BEV_Z), jnp.ones_like(xc)], axis=1)
    inds, mask, imtab = _tc_indices(intrin[0], E.reshape(N, 4, 4), pts4,
                                    images.reshape(N, C, H * W))
    idx_w = inds.reshape(NC, NS, IDX_ROWS, SUB)
    feats = _sc_gather(imtab.reshape(N * H * W, C), idx_w)
    return feats, mask.reshape(B, N, K, 1)
